# baseline (speedup 1.0000x reference)
import functools

import jax
import jax.numpy as jnp
from jax import lax
from jax.experimental import pallas as pl
from jax.experimental.pallas import tpu as pltpu
from jax.experimental.pallas import tpu_sc as plsc

N = 10000
E = 320000
D = 128
H = 128

NW = 32
K = 80
NCH = 128
G = 32
NG0 = 2
NG1 = 6
TOTAL_CH = 2 * 16 * NCH
EW = K * NCH
E_PAD = NW * EW
N_PAD = 10240
ROWS_PER_SUB = N_PAD // 16
CNT_W = 16
CBLK = 64


_INV_SQRT2 = 0.7071067811865476


def _gelu(x):
    return 0.5 * x * (1.0 + lax.erf(x * _INV_SQRT2))


def _ffn_kernel(x_ref, w1_ref, c1_ref, w2_ref, c2_ref, o_ref):
    x = x_ref[...]
    z = jnp.dot(x, w1_ref[...], preferred_element_type=jnp.float32) + c1_ref[...]
    y = _gelu(z)
    z2 = jnp.dot(y, w2_ref[...], preferred_element_type=jnp.float32) + c2_ref[...]
    o_ref[...] = _gelu(z2)


def _prep_ffn(x, w1, c1, w2, c2, block_rows=2000):
    n = x.shape[0]
    grid = (n + block_rows - 1) // block_rows
    return pl.pallas_call(
        _ffn_kernel,
        grid=(grid,),
        in_specs=[
            pl.BlockSpec((block_rows, D), lambda i: (i, 0)),
            pl.BlockSpec((D, H), lambda i: (0, 0)),
            pl.BlockSpec((1, H), lambda i: (0, 0)),
            pl.BlockSpec((H, H), lambda i: (0, 0)),
            pl.BlockSpec((1, H), lambda i: (0, 0)),
        ],
        out_specs=pl.BlockSpec((block_rows, H), lambda i: (i, 0)),
        out_shape=jax.ShapeDtypeStruct((n, H), jnp.float32),
    )(x, w1, c1, w2, c2)


def _sc_body(table_hbm, dst_hbm, src_hbm, w_hbm, ones_hbm, sums_hbm, cnts_hbm,
             dst_v, src_v, w_v, rows_a, rows_b, ones_v, zcnt_v,
             sg_a, sg_b, ss_a, ss_b, sc_c, acc_sh, cnt_sh):
    cid = lax.axis_index("c")
    sid = lax.axis_index("s")
    row0 = sid * ROWS_PER_SUB
    chunk0 = jnp.where(cid == 0, sid * (NG0 * G),
                       16 * (NG0 * G) + sid * (NG1 * G))
    ngrp = jnp.where(cid == 0, NG0, NG1)
    rows = (rows_a, rows_b)
    sg = (sg_a, sg_b)
    ss = (ss_a, ss_b)

    def zrow_body(i, c):
        for r in range(H // 16):
            rows_a[i, pl.ds(r * 16, 16)] = jnp.zeros((16,), jnp.float32)
        return c

    lax.fori_loop(0, K, zrow_body, 0)

    def zcnt_body(i, c):
        zcnt_v[i, pl.ds(0, CNT_W)] = jnp.zeros((CNT_W,), jnp.float32)
        return c

    lax.fori_loop(0, CBLK, zcnt_body, 0)

    for t in range(ROWS_PER_SUB // 64):
        pltpu.sync_copy(rows_a.at[pl.ds(0, 64)],
                        acc_sh.at[pl.ds(row0 + t * 64, 64)])
    for t in range(ROWS_PER_SUB // CBLK):
        pltpu.sync_copy(zcnt_v, cnt_sh.at[pl.ds(row0 + t * CBLK, CBLK)])

    pltpu.sync_copy(ones_hbm, ones_v)
    plsc.subcore_barrier()

    def edge_chunk_scale(jj):
        buf = rows[jj % 2]

        def edge_body(i, c2):
            grp = w_v[jj, pl.ds((i // 16) * 16, 16)]
            wv = lax.gather(
                grp, jnp.full((16, 1), i % 16, jnp.int32),
                lax.GatherDimensionNumbers(offset_dims=(),
                                           collapsed_slice_dims=(0,),
                                           start_index_map=(0,)),
                (1,), mode=lax.GatherScatterMode.PROMISE_IN_BOUNDS)
            for r in range(H // 16):
                sl = (i, pl.ds(r * 16, 16))
                buf[sl] = buf[sl] * wv
            return c2

        lax.fori_loop(0, K, edge_body, 0, unroll=2)

    def group_body(g, carry):
        c0 = chunk0 + g * G
        pltpu.sync_copy(dst_hbm.at[pl.ds(c0, G)], dst_v)
        pltpu.sync_copy(src_hbm.at[pl.ds(c0, G)], src_v)
        pltpu.sync_copy(w_hbm.at[pl.ds(c0, G)], w_v)

        pltpu.async_copy(table_hbm.at[src_v.at[0]], rows[0], sg[0])
        cnt_pend = []
        for j in range(G):
            b = j % 2
            nb = (j + 1) % 2
            if j + 1 < G:
                if j >= 1:
                    pltpu.make_async_copy(rows[nb], acc_sh.at[dst_v.at[j - 1]],
                                          ss[nb]).wait()
                pltpu.async_copy(table_hbm.at[src_v.at[j + 1]], rows[nb],
                                 sg[nb])
            pltpu.make_async_copy(table_hbm.at[src_v.at[j]], rows[b],
                                  sg[b]).wait()
            edge_chunk_scale(j)
            pltpu.async_copy(rows[b], acc_sh.at[dst_v.at[j]], ss[b], add=True)
            if cnt_pend:
                cnt_pend.pop().wait()
            cnt_pend.append(
                pltpu.async_copy(ones_v, cnt_sh.at[dst_v.at[j]], sc_c,
                                 add=True))
        pltpu.make_async_copy(rows[(G - 2) % 2], acc_sh.at[dst_v.at[G - 2]],
                              ss[(G - 2) % 2]).wait()
        pltpu.make_async_copy(rows[(G - 1) % 2], acc_sh.at[dst_v.at[G - 1]],
                              ss[(G - 1) % 2]).wait()
        cnt_pend.pop().wait()
        return carry

    lax.fori_loop(0, ngrp, group_body, 0)
    plsc.subcore_barrier()

    for t in range(ROWS_PER_SUB // 64):
        pltpu.sync_copy(acc_sh.at[pl.ds(row0 + t * 64, 64)],
                        sums_hbm.at[cid, sid * (ROWS_PER_SUB // 64) + t])
    for t in range(ROWS_PER_SUB // CBLK):
        pltpu.sync_copy(cnt_sh.at[pl.ds(row0 + t * CBLK, CBLK)],
                        cnts_hbm.at[cid, sid * (ROWS_PER_SUB // CBLK) + t])


def _sc_segment_sum(table, dst3, src3, w3, ones):
    mesh = plsc.VectorSubcoreMesh(core_axis_name="c", subcore_axis_name="s")
    f = pl.kernel(
        _sc_body,
        out_type=[
            jax.ShapeDtypeStruct((2, N_PAD // 64, 64, H), jnp.float32),
            jax.ShapeDtypeStruct((2, N_PAD // CBLK, CBLK, CNT_W), jnp.float32),
        ],
        mesh=mesh,
        compiler_params=pltpu.CompilerParams(use_tc_tiling_on_sc=False),
        scratch_types=[
            pltpu.VMEM((G, K), jnp.int32),
            pltpu.VMEM((G, K), jnp.int32),
            pltpu.VMEM((G, K), jnp.float32),
            pltpu.VMEM((K, H), jnp.float32),
            pltpu.VMEM((K, H), jnp.float32),
            pltpu.VMEM((K, CNT_W), jnp.float32),
            pltpu.VMEM((CBLK, CNT_W), jnp.float32),
            pltpu.SemaphoreType.DMA,
            pltpu.SemaphoreType.DMA,
            pltpu.SemaphoreType.DMA,
            pltpu.SemaphoreType.DMA,
            pltpu.SemaphoreType.DMA,
            pltpu.VMEM_SHARED((N_PAD, H), jnp.float32),
            pltpu.VMEM_SHARED((N_PAD, CNT_W), jnp.float32),
        ],
    )
    sums, cnts = f(table, dst3, src3, w3, ones)
    return sums.reshape(2, N_PAD, H), cnts.reshape(2, N_PAD, CNT_W)


def _upd_kernel(x_ref, sums_ref, cnts_ref, w1x_ref, w1a_ref, c1_ref,
                w2_ref, c2_ref, o_ref):
    s = sums_ref[0] + sums_ref[1]
    c = cnts_ref[0, :, 0:1] + cnts_ref[1, :, 0:1]
    agg = s / jnp.maximum(c, 1.0)
    x = x_ref[...]
    z = (jnp.dot(x, w1x_ref[...], preferred_element_type=jnp.float32)
         + jnp.dot(agg, w1a_ref[...], preferred_element_type=jnp.float32)
         + c1_ref[...])
    y = _gelu(z)
    z2 = jnp.dot(y, w2_ref[...], preferred_element_type=jnp.float32) + c2_ref[...]
    o_ref[...] = _gelu(z2)


def _update_ffn(x, sums, cnts, w1x, w1a, c1, w2, c2, block_rows=2000):
    n = x.shape[0]
    grid = (n + block_rows - 1) // block_rows
    return pl.pallas_call(
        _upd_kernel,
        grid=(grid,),
        in_specs=[
            pl.BlockSpec((block_rows, D), lambda i: (i, 0)),
            pl.BlockSpec((2, block_rows, H), lambda i: (0, i, 0)),
            pl.BlockSpec((2, block_rows, CNT_W), lambda i: (0, i, 0)),
            pl.BlockSpec((D, H), lambda i: (0, 0)),
            pl.BlockSpec((H, H), lambda i: (0, 0)),
            pl.BlockSpec((1, H), lambda i: (0, 0)),
            pl.BlockSpec((H, H), lambda i: (0, 0)),
            pl.BlockSpec((1, H), lambda i: (0, 0)),
        ],
        out_specs=pl.BlockSpec((block_rows, H), lambda i: (i, 0)),
        out_shape=jax.ShapeDtypeStruct((n, H), jnp.float32),
    )(x, sums, cnts, w1x, w1a, c1, w2, c2)


def _fold_bn(g, b, m, v, w, bias):
    s = g / jnp.sqrt(v + 1e-3)
    t = b - m * s
    return w * s[:, None], (t @ w + bias)[None, :]


def kernel(node_representations, edges, edge_weights,
           prep_bn1_g, prep_bn1_b, prep_bn1_m, prep_bn1_v, prep_w1, prep_b1,
           prep_bn2_g, prep_bn2_b, prep_bn2_m, prep_bn2_v, prep_w2, prep_b2,
           upd_bn1_g, upd_bn1_b, upd_bn1_m, upd_bn1_v, upd_w1, upd_b1,
           upd_bn2_g, upd_bn2_b, upd_bn2_m, upd_bn2_v, upd_w2, upd_b2):
    x = node_representations

    pw1, pc1 = _fold_bn(prep_bn1_g, prep_bn1_b, prep_bn1_m, prep_bn1_v,
                        prep_w1, prep_b1)
    pw2, pc2 = _fold_bn(prep_bn2_g, prep_bn2_b, prep_bn2_m, prep_bn2_v,
                        prep_w2, prep_b2)
    uw1, uc1 = _fold_bn(upd_bn1_g, upd_bn1_b, upd_bn1_m, upd_bn1_v,
                        upd_w1, upd_b1)
    uw2, uc2 = _fold_bn(upd_bn2_g, upd_bn2_b, upd_bn2_m, upd_bn2_v,
                        upd_w2, upd_b2)
    uw1x, uw1a = uw1[:D], uw1[D:]

    msg = _prep_ffn(x, pw1, pc1, pw2, pc2)

    pad = E_PAD - E
    dst = jnp.concatenate([edges[0], jnp.full((pad,), N, jnp.int32)])
    src = jnp.concatenate([edges[1], jnp.zeros((pad,), jnp.int32)])
    w = jnp.concatenate([edge_weights, jnp.zeros((pad,), jnp.float32)])
    dst3 = dst.reshape(TOTAL_CH, K)
    src3 = src.reshape(TOTAL_CH, K)
    w3 = w.reshape(TOTAL_CH, K)

    ones = jnp.ones((K, CNT_W), jnp.float32)

    sums, cnts = _sc_segment_sum(msg, dst3, src3, w3, ones)

    return _update_ffn(x, sums, cnts, uw1x, uw1a, uc1, uw2, uc2)


# speedup vs baseline: 1.2321x; 1.2321x over previous
import functools

import jax
import jax.numpy as jnp
from jax import lax
from jax.experimental import pallas as pl
from jax.experimental.pallas import tpu as pltpu
from jax.experimental.pallas import tpu_sc as plsc

N = 10000
E = 320000
D = 128
H = 128

NW = 32
K = 80
NCH = 128
G = 32
NG0 = 6
NG1 = 2
TOTAL_CH = 2 * 16 * NCH
EW = K * NCH
E_PAD = NW * EW
N_PAD = 10240
ROWS_PER_SUB = N_PAD // 16
CNT_W = 16
CBLK = 64


_INV_SQRT2 = 0.7071067811865476


def _gelu(x):
    return 0.5 * x * (1.0 + lax.erf(x * _INV_SQRT2))


def _ffn_kernel(x_ref, w1_ref, c1_ref, w2_ref, c2_ref, o_ref):
    x = x_ref[...]
    z = jnp.dot(x, w1_ref[...], preferred_element_type=jnp.float32) + c1_ref[...]
    y = _gelu(z)
    z2 = jnp.dot(y, w2_ref[...], preferred_element_type=jnp.float32) + c2_ref[...]
    o_ref[...] = _gelu(z2)


def _prep_ffn(x, w1, c1, w2, c2, block_rows=2000):
    n = x.shape[0]
    grid = (n + block_rows - 1) // block_rows
    return pl.pallas_call(
        _ffn_kernel,
        grid=(grid,),
        in_specs=[
            pl.BlockSpec((block_rows, D), lambda i: (i, 0)),
            pl.BlockSpec((D, H), lambda i: (0, 0)),
            pl.BlockSpec((1, H), lambda i: (0, 0)),
            pl.BlockSpec((H, H), lambda i: (0, 0)),
            pl.BlockSpec((1, H), lambda i: (0, 0)),
        ],
        out_specs=pl.BlockSpec((block_rows, H), lambda i: (i, 0)),
        out_shape=jax.ShapeDtypeStruct((n, H), jnp.float32),
    )(x, w1, c1, w2, c2)


def _sc_body(table_hbm, dst_hbm, src_hbm, w_hbm, ones_hbm, sums_hbm, cnts_hbm,
             dst_v, src_v, w_v, rows_a, rows_b, ones_v, zcnt_v,
             sg_a, sg_b, ss_a, ss_b, sc_c, acc_sh, cnt_sh):
    cid = lax.axis_index("c")
    sid = lax.axis_index("s")
    row0 = sid * ROWS_PER_SUB
    chunk0 = jnp.where(cid == 0, sid * (NG0 * G),
                       16 * (NG0 * G) + sid * (NG1 * G))
    ngrp = jnp.where(cid == 0, NG0, NG1)
    rows = (rows_a, rows_b)
    sg = (sg_a, sg_b)
    ss = (ss_a, ss_b)

    def zrow_body(i, c):
        for r in range(H // 16):
            rows_a[i, pl.ds(r * 16, 16)] = jnp.zeros((16,), jnp.float32)
        return c

    lax.fori_loop(0, K, zrow_body, 0)

    def zcnt_body(i, c):
        zcnt_v[i, pl.ds(0, CNT_W)] = jnp.zeros((CNT_W,), jnp.float32)
        return c

    lax.fori_loop(0, CBLK, zcnt_body, 0)

    for t in range(ROWS_PER_SUB // 64):
        pltpu.sync_copy(rows_a.at[pl.ds(0, 64)],
                        acc_sh.at[pl.ds(row0 + t * 64, 64)])
    for t in range(ROWS_PER_SUB // CBLK):
        pltpu.sync_copy(zcnt_v, cnt_sh.at[pl.ds(row0 + t * CBLK, CBLK)])

    pltpu.sync_copy(ones_hbm, ones_v)
    plsc.subcore_barrier()

    def edge_chunk_scale(jj):
        buf = rows[jj % 2]

        def edge_body(i, c2):
            grp = w_v[jj, pl.ds((i // 16) * 16, 16)]
            wv = lax.gather(
                grp, jnp.full((16, 1), i % 16, jnp.int32),
                lax.GatherDimensionNumbers(offset_dims=(),
                                           collapsed_slice_dims=(0,),
                                           start_index_map=(0,)),
                (1,), mode=lax.GatherScatterMode.PROMISE_IN_BOUNDS)
            for r in range(H // 16):
                sl = (i, pl.ds(r * 16, 16))
                buf[sl] = buf[sl] * wv
            return c2

        lax.fori_loop(0, K, edge_body, 0, unroll=2)

    def group_body(g, carry):
        c0 = chunk0 + g * G
        pltpu.sync_copy(dst_hbm.at[pl.ds(c0, G)], dst_v)
        pltpu.sync_copy(src_hbm.at[pl.ds(c0, G)], src_v)
        pltpu.sync_copy(w_hbm.at[pl.ds(c0, G)], w_v)

        pltpu.async_copy(table_hbm.at[src_v.at[0]], rows[0], sg[0])
        cnt_pend = []
        for j in range(G):
            b = j % 2
            nb = (j + 1) % 2
            if j + 1 < G:
                if j >= 1:
                    pltpu.make_async_copy(rows[nb], acc_sh.at[dst_v.at[j - 1]],
                                          ss[nb]).wait()
                pltpu.async_copy(table_hbm.at[src_v.at[j + 1]], rows[nb],
                                 sg[nb])
            pltpu.make_async_copy(table_hbm.at[src_v.at[j]], rows[b],
                                  sg[b]).wait()
            edge_chunk_scale(j)
            pltpu.async_copy(rows[b], acc_sh.at[dst_v.at[j]], ss[b], add=True)
            if cnt_pend:
                cnt_pend.pop().wait()
            cnt_pend.append(
                pltpu.async_copy(ones_v, cnt_sh.at[dst_v.at[j]], sc_c,
                                 add=True))
        pltpu.make_async_copy(rows[(G - 2) % 2], acc_sh.at[dst_v.at[G - 2]],
                              ss[(G - 2) % 2]).wait()
        pltpu.make_async_copy(rows[(G - 1) % 2], acc_sh.at[dst_v.at[G - 1]],
                              ss[(G - 1) % 2]).wait()
        cnt_pend.pop().wait()
        return carry

    lax.fori_loop(0, ngrp, group_body, 0)
    plsc.subcore_barrier()

    for t in range(ROWS_PER_SUB // 64):
        pltpu.sync_copy(acc_sh.at[pl.ds(row0 + t * 64, 64)],
                        sums_hbm.at[cid, sid * (ROWS_PER_SUB // 64) + t])
    for t in range(ROWS_PER_SUB // CBLK):
        pltpu.sync_copy(cnt_sh.at[pl.ds(row0 + t * CBLK, CBLK)],
                        cnts_hbm.at[cid, sid * (ROWS_PER_SUB // CBLK) + t])


def _sc_segment_sum(table, dst3, src3, w3, ones):
    mesh = plsc.VectorSubcoreMesh(core_axis_name="c", subcore_axis_name="s")
    f = pl.kernel(
        _sc_body,
        out_type=[
            jax.ShapeDtypeStruct((2, N_PAD // 64, 64, H), jnp.float32),
            jax.ShapeDtypeStruct((2, N_PAD // CBLK, CBLK, CNT_W), jnp.float32),
        ],
        mesh=mesh,
        compiler_params=pltpu.CompilerParams(use_tc_tiling_on_sc=False),
        scratch_types=[
            pltpu.VMEM((G, K), jnp.int32),
            pltpu.VMEM((G, K), jnp.int32),
            pltpu.VMEM((G, K), jnp.float32),
            pltpu.VMEM((K, H), jnp.float32),
            pltpu.VMEM((K, H), jnp.float32),
            pltpu.VMEM((K, CNT_W), jnp.float32),
            pltpu.VMEM((CBLK, CNT_W), jnp.float32),
            pltpu.SemaphoreType.DMA,
            pltpu.SemaphoreType.DMA,
            pltpu.SemaphoreType.DMA,
            pltpu.SemaphoreType.DMA,
            pltpu.SemaphoreType.DMA,
            pltpu.VMEM_SHARED((N_PAD, H), jnp.float32),
            pltpu.VMEM_SHARED((N_PAD, CNT_W), jnp.float32),
        ],
    )
    sums, cnts = f(table, dst3, src3, w3, ones)
    return sums.reshape(2, N_PAD, H), cnts.reshape(2, N_PAD, CNT_W)


def _upd_kernel(x_ref, sums_ref, cnts_ref, w1x_ref, w1a_ref, c1_ref,
                w2_ref, c2_ref, o_ref):
    s = sums_ref[0] + sums_ref[1]
    c = cnts_ref[0, :, 0:1] + cnts_ref[1, :, 0:1]
    agg = s / jnp.maximum(c, 1.0)
    x = x_ref[...]
    z = (jnp.dot(x, w1x_ref[...], preferred_element_type=jnp.float32)
         + jnp.dot(agg, w1a_ref[...], preferred_element_type=jnp.float32)
         + c1_ref[...])
    y = _gelu(z)
    z2 = jnp.dot(y, w2_ref[...], preferred_element_type=jnp.float32) + c2_ref[...]
    o_ref[...] = _gelu(z2)


def _update_ffn(x, sums, cnts, w1x, w1a, c1, w2, c2, block_rows=2000):
    n = x.shape[0]
    grid = (n + block_rows - 1) // block_rows
    return pl.pallas_call(
        _upd_kernel,
        grid=(grid,),
        in_specs=[
            pl.BlockSpec((block_rows, D), lambda i: (i, 0)),
            pl.BlockSpec((2, block_rows, H), lambda i: (0, i, 0)),
            pl.BlockSpec((2, block_rows, CNT_W), lambda i: (0, i, 0)),
            pl.BlockSpec((D, H), lambda i: (0, 0)),
            pl.BlockSpec((H, H), lambda i: (0, 0)),
            pl.BlockSpec((1, H), lambda i: (0, 0)),
            pl.BlockSpec((H, H), lambda i: (0, 0)),
            pl.BlockSpec((1, H), lambda i: (0, 0)),
        ],
        out_specs=pl.BlockSpec((block_rows, H), lambda i: (i, 0)),
        out_shape=jax.ShapeDtypeStruct((n, H), jnp.float32),
    )(x, sums, cnts, w1x, w1a, c1, w2, c2)


def _fold_bn(g, b, m, v, w, bias):
    s = g / jnp.sqrt(v + 1e-3)
    t = b - m * s
    return w * s[:, None], (t @ w + bias)[None, :]


def kernel(node_representations, edges, edge_weights,
           prep_bn1_g, prep_bn1_b, prep_bn1_m, prep_bn1_v, prep_w1, prep_b1,
           prep_bn2_g, prep_bn2_b, prep_bn2_m, prep_bn2_v, prep_w2, prep_b2,
           upd_bn1_g, upd_bn1_b, upd_bn1_m, upd_bn1_v, upd_w1, upd_b1,
           upd_bn2_g, upd_bn2_b, upd_bn2_m, upd_bn2_v, upd_w2, upd_b2):
    x = node_representations

    pw1, pc1 = _fold_bn(prep_bn1_g, prep_bn1_b, prep_bn1_m, prep_bn1_v,
                        prep_w1, prep_b1)
    pw2, pc2 = _fold_bn(prep_bn2_g, prep_bn2_b, prep_bn2_m, prep_bn2_v,
                        prep_w2, prep_b2)
    uw1, uc1 = _fold_bn(upd_bn1_g, upd_bn1_b, upd_bn1_m, upd_bn1_v,
                        upd_w1, upd_b1)
    uw2, uc2 = _fold_bn(upd_bn2_g, upd_bn2_b, upd_bn2_m, upd_bn2_v,
                        upd_w2, upd_b2)
    uw1x, uw1a = uw1[:D], uw1[D:]

    msg = _prep_ffn(x, pw1, pc1, pw2, pc2)

    pad = E_PAD - E
    dst = jnp.concatenate([edges[0], jnp.full((pad,), N, jnp.int32)])
    src = jnp.concatenate([edges[1], jnp.zeros((pad,), jnp.int32)])
    w = jnp.concatenate([edge_weights, jnp.zeros((pad,), jnp.float32)])
    dst3 = dst.reshape(TOTAL_CH, K)
    src3 = src.reshape(TOTAL_CH, K)
    w3 = w.reshape(TOTAL_CH, K)

    ones = jnp.ones((K, CNT_W), jnp.float32)

    sums, cnts = _sc_segment_sum(msg, dst3, src3, w3, ones)

    return _update_ffn(x, sums, cnts, uw1x, uw1a, uc1, uw2, uc2)


# speedup vs baseline: 1.2824x; 1.0408x over previous
import functools

import jax
import jax.numpy as jnp
from jax import lax
from jax.experimental import pallas as pl
from jax.experimental.pallas import tpu as pltpu
from jax.experimental.pallas import tpu_sc as plsc

N = 10000
E = 320000
D = 128
H = 128

NW = 32
K = 80
NCH = 128
G = 32
NG0 = 6
NG1 = 2
TOTAL_CH = 2 * 16 * NCH
EW = K * NCH
E_PAD = NW * EW
N_PAD = 10240
ROWS_PER_SUB = N_PAD // 16
CNT_W = 16
CBLK = 64


_INV_SQRT2 = 0.7071067811865476


def _gelu(x):
    return 0.5 * x * (1.0 + lax.erf(x * _INV_SQRT2))


def _ffn_kernel(x_ref, w1_ref, c1_ref, w2_ref, c2_ref, o_ref):
    x = x_ref[...]
    z = jnp.dot(x, w1_ref[...], preferred_element_type=jnp.float32) + c1_ref[...]
    y = _gelu(z)
    z2 = jnp.dot(y, w2_ref[...], preferred_element_type=jnp.float32) + c2_ref[...]
    o_ref[...] = _gelu(z2)


def _prep_ffn(x, w1, c1, w2, c2, block_rows=2000):
    n = x.shape[0]
    grid = (n + block_rows - 1) // block_rows
    return pl.pallas_call(
        _ffn_kernel,
        grid=(grid,),
        in_specs=[
            pl.BlockSpec((block_rows, D), lambda i: (i, 0)),
            pl.BlockSpec((D, H), lambda i: (0, 0)),
            pl.BlockSpec((1, H), lambda i: (0, 0)),
            pl.BlockSpec((H, H), lambda i: (0, 0)),
            pl.BlockSpec((1, H), lambda i: (0, 0)),
        ],
        out_specs=pl.BlockSpec((block_rows, H), lambda i: (i, 0)),
        out_shape=jax.ShapeDtypeStruct((n, H), jnp.float32),
    )(x, w1, c1, w2, c2)


def _sc_body(table_hbm, dst_hbm, src_hbm, w_hbm, ones_hbm, sums_hbm, cnts_hbm,
             dst_v, src_v, w_v, rows_a, rows_b, ones_v, zcnt_v,
             sg_a, sg_b, ss_a, ss_b, sc_c, acc_sh, cnt_sh):
    cid = lax.axis_index("c")
    sid = lax.axis_index("s")
    row0 = sid * ROWS_PER_SUB
    chunk0 = jnp.where(cid == 0, sid * (NG0 * G),
                       16 * (NG0 * G) + sid * (NG1 * G))
    ngrp = jnp.where(cid == 0, NG0, NG1)
    rows = (rows_a, rows_b)
    sg = (sg_a, sg_b)
    ss = (ss_a, ss_b)

    def zrow_body(i, c):
        for r in range(H // 16):
            rows_a[i, pl.ds(r * 16, 16)] = jnp.zeros((16,), jnp.float32)
        return c

    lax.fori_loop(0, K, zrow_body, 0)

    def zcnt_body(i, c):
        zcnt_v[i, pl.ds(0, CNT_W)] = jnp.zeros((CNT_W,), jnp.float32)
        return c

    lax.fori_loop(0, CBLK, zcnt_body, 0)

    for t in range(ROWS_PER_SUB // 64):
        pltpu.sync_copy(rows_a.at[pl.ds(0, 64)],
                        acc_sh.at[pl.ds(row0 + t * 64, 64)])
    for t in range(ROWS_PER_SUB // CBLK):
        pltpu.sync_copy(zcnt_v, cnt_sh.at[pl.ds(row0 + t * CBLK, CBLK)])

    pltpu.sync_copy(ones_hbm, ones_v)
    plsc.subcore_barrier()

    def edge_chunk_scale(jj):
        buf = rows[jj % 2]
        dn = lax.GatherDimensionNumbers(offset_dims=(),
                                        collapsed_slice_dims=(0,),
                                        start_index_map=(0,))

        @plsc.parallel_loop(0, K, unroll=2)
        def _edge(i):
            grp = w_v[jj, pl.ds((i // 16) * 16, 16)]
            wv = lax.gather(grp, jnp.full((16, 1), i % 16, jnp.int32), dn,
                            (1,), mode=lax.GatherScatterMode.PROMISE_IN_BOUNDS)
            for r in range(H // 16):
                sl = (i, pl.ds(r * 16, 16))
                buf[sl] = buf[sl] * wv

    def group_body(g, carry):
        c0 = chunk0 + g * G
        pltpu.sync_copy(dst_hbm.at[pl.ds(c0, G)], dst_v)
        pltpu.sync_copy(src_hbm.at[pl.ds(c0, G)], src_v)
        pltpu.sync_copy(w_hbm.at[pl.ds(c0, G)], w_v)

        pltpu.async_copy(table_hbm.at[src_v.at[0]], rows[0], sg[0])
        cnt_pend = []
        for j in range(G):
            b = j % 2
            nb = (j + 1) % 2
            if j + 1 < G:
                if j >= 1:
                    pltpu.make_async_copy(rows[nb], acc_sh.at[dst_v.at[j - 1]],
                                          ss[nb]).wait()
                pltpu.async_copy(table_hbm.at[src_v.at[j + 1]], rows[nb],
                                 sg[nb])
            pltpu.make_async_copy(table_hbm.at[src_v.at[j]], rows[b],
                                  sg[b]).wait()
            edge_chunk_scale(j)
            pltpu.async_copy(rows[b], acc_sh.at[dst_v.at[j]], ss[b], add=True)
            if cnt_pend:
                cnt_pend.pop().wait()
            cnt_pend.append(
                pltpu.async_copy(ones_v, cnt_sh.at[dst_v.at[j]], sc_c,
                                 add=True))
        pltpu.make_async_copy(rows[(G - 2) % 2], acc_sh.at[dst_v.at[G - 2]],
                              ss[(G - 2) % 2]).wait()
        pltpu.make_async_copy(rows[(G - 1) % 2], acc_sh.at[dst_v.at[G - 1]],
                              ss[(G - 1) % 2]).wait()
        cnt_pend.pop().wait()
        return carry

    lax.fori_loop(0, ngrp, group_body, 0)
    plsc.subcore_barrier()

    for t in range(ROWS_PER_SUB // 64):
        pltpu.sync_copy(acc_sh.at[pl.ds(row0 + t * 64, 64)],
                        sums_hbm.at[cid, sid * (ROWS_PER_SUB // 64) + t])
    for t in range(ROWS_PER_SUB // CBLK):
        pltpu.sync_copy(cnt_sh.at[pl.ds(row0 + t * CBLK, CBLK)],
                        cnts_hbm.at[cid, sid * (ROWS_PER_SUB // CBLK) + t])


def _sc_segment_sum(table, dst3, src3, w3, ones):
    mesh = plsc.VectorSubcoreMesh(core_axis_name="c", subcore_axis_name="s")
    f = pl.kernel(
        _sc_body,
        out_type=[
            jax.ShapeDtypeStruct((2, N_PAD // 64, 64, H), jnp.float32),
            jax.ShapeDtypeStruct((2, N_PAD // CBLK, CBLK, CNT_W), jnp.float32),
        ],
        mesh=mesh,
        compiler_params=pltpu.CompilerParams(use_tc_tiling_on_sc=False),
        scratch_types=[
            pltpu.VMEM((G, K), jnp.int32),
            pltpu.VMEM((G, K), jnp.int32),
            pltpu.VMEM((G, K), jnp.float32),
            pltpu.VMEM((K, H), jnp.float32),
            pltpu.VMEM((K, H), jnp.float32),
            pltpu.VMEM((K, CNT_W), jnp.float32),
            pltpu.VMEM((CBLK, CNT_W), jnp.float32),
            pltpu.SemaphoreType.DMA,
            pltpu.SemaphoreType.DMA,
            pltpu.SemaphoreType.DMA,
            pltpu.SemaphoreType.DMA,
            pltpu.SemaphoreType.DMA,
            pltpu.VMEM_SHARED((N_PAD, H), jnp.float32),
            pltpu.VMEM_SHARED((N_PAD, CNT_W), jnp.float32),
        ],
    )
    sums, cnts = f(table, dst3, src3, w3, ones)
    return sums.reshape(2, N_PAD, H), cnts.reshape(2, N_PAD, CNT_W)


def _upd_kernel(x_ref, sums_ref, cnts_ref, w1x_ref, w1a_ref, c1_ref,
                w2_ref, c2_ref, o_ref):
    s = sums_ref[0] + sums_ref[1]
    c = cnts_ref[0, :, 0:1] + cnts_ref[1, :, 0:1]
    agg = s / jnp.maximum(c, 1.0)
    x = x_ref[...]
    z = (jnp.dot(x, w1x_ref[...], preferred_element_type=jnp.float32)
         + jnp.dot(agg, w1a_ref[...], preferred_element_type=jnp.float32)
         + c1_ref[...])
    y = _gelu(z)
    z2 = jnp.dot(y, w2_ref[...], preferred_element_type=jnp.float32) + c2_ref[...]
    o_ref[...] = _gelu(z2)


def _update_ffn(x, sums, cnts, w1x, w1a, c1, w2, c2, block_rows=2000):
    n = x.shape[0]
    grid = (n + block_rows - 1) // block_rows
    return pl.pallas_call(
        _upd_kernel,
        grid=(grid,),
        in_specs=[
            pl.BlockSpec((block_rows, D), lambda i: (i, 0)),
            pl.BlockSpec((2, block_rows, H), lambda i: (0, i, 0)),
            pl.BlockSpec((2, block_rows, CNT_W), lambda i: (0, i, 0)),
            pl.BlockSpec((D, H), lambda i: (0, 0)),
            pl.BlockSpec((H, H), lambda i: (0, 0)),
            pl.BlockSpec((1, H), lambda i: (0, 0)),
            pl.BlockSpec((H, H), lambda i: (0, 0)),
            pl.BlockSpec((1, H), lambda i: (0, 0)),
        ],
        out_specs=pl.BlockSpec((block_rows, H), lambda i: (i, 0)),
        out_shape=jax.ShapeDtypeStruct((n, H), jnp.float32),
    )(x, sums, cnts, w1x, w1a, c1, w2, c2)


def _fold_bn(g, b, m, v, w, bias):
    s = g / jnp.sqrt(v + 1e-3)
    t = b - m * s
    return w * s[:, None], (t @ w + bias)[None, :]


def kernel(node_representations, edges, edge_weights,
           prep_bn1_g, prep_bn1_b, prep_bn1_m, prep_bn1_v, prep_w1, prep_b1,
           prep_bn2_g, prep_bn2_b, prep_bn2_m, prep_bn2_v, prep_w2, prep_b2,
           upd_bn1_g, upd_bn1_b, upd_bn1_m, upd_bn1_v, upd_w1, upd_b1,
           upd_bn2_g, upd_bn2_b, upd_bn2_m, upd_bn2_v, upd_w2, upd_b2):
    x = node_representations

    pw1, pc1 = _fold_bn(prep_bn1_g, prep_bn1_b, prep_bn1_m, prep_bn1_v,
                        prep_w1, prep_b1)
    pw2, pc2 = _fold_bn(prep_bn2_g, prep_bn2_b, prep_bn2_m, prep_bn2_v,
                        prep_w2, prep_b2)
    uw1, uc1 = _fold_bn(upd_bn1_g, upd_bn1_b, upd_bn1_m, upd_bn1_v,
                        upd_w1, upd_b1)
    uw2, uc2 = _fold_bn(upd_bn2_g, upd_bn2_b, upd_bn2_m, upd_bn2_v,
                        upd_w2, upd_b2)
    uw1x, uw1a = uw1[:D], uw1[D:]

    msg = _prep_ffn(x, pw1, pc1, pw2, pc2)

    pad = E_PAD - E
    dst = jnp.concatenate([edges[0], jnp.full((pad,), N, jnp.int32)])
    src = jnp.concatenate([edges[1], jnp.zeros((pad,), jnp.int32)])
    w = jnp.concatenate([edge_weights, jnp.zeros((pad,), jnp.float32)])
    dst3 = dst.reshape(TOTAL_CH, K)
    src3 = src.reshape(TOTAL_CH, K)
    w3 = w.reshape(TOTAL_CH, K)

    ones = jnp.ones((K, CNT_W), jnp.float32)

    sums, cnts = _sc_segment_sum(msg, dst3, src3, w3, ones)

    return _update_ffn(x, sums, cnts, uw1x, uw1a, uc1, uw2, uc2)
